# baseline (speedup 1.0000x reference)
import functools

import jax
import jax.numpy as jnp
from jax import lax
from jax.experimental import pallas as pl
from jax.experimental.pallas import tpu as pltpu
from jax.experimental.pallas import tpu_sc as plsc

VOCAB = 1000000
DIM = 32
BATCH = 16384
HIST = 50
N = BATCH * HIST

NUM_CORES = 2
NUM_SUBCORES = 16
NW = NUM_CORES * NUM_SUBCORES

PER_W = N // NW
CHUNK = 800
NCHUNK = PER_W // CHUNK
NBUF = 4

NBLK = (VOCAB + 127) // 128
FULL_K = (NBLK - 1) // NW
BPD = 128 * DIM

_mesh = plsc.VectorSubcoreMesh(core_axis_name="c", subcore_axis_name="s")


def _worker_id():
    return lax.axis_index("s") * NUM_CORES + lax.axis_index("c")


@functools.partial(
    pl.kernel,
    mesh=_mesh,
    out_type=jax.ShapeDtypeStruct((VOCAB * DIM,), jnp.float32),
    scratch_types=[
        pltpu.VMEM((2, DIM, 128), jnp.float32),
        pltpu.VMEM((2, BPD), jnp.float32),
        pltpu.SemaphoreType.DMA((2,)),
        pltpu.SemaphoreType.DMA((2,)),
    ],
    compiler_params=pltpu.CompilerParams(needs_layout_passes=False),
)
def _transpose_kernel(tt_hbm, out_hbm, in_v, out_v, gsem, osem):
    w = _worker_id()

    def in_slice(it):
        return tt_hbm.at[pl.ds(0, DIM), pl.ds(it * 128, 128)]

    def fire_in(it, b):
        pltpu.async_copy(in_slice(it), in_v.at[b], gsem.at[b])

    def wait_in(b):
        pltpu.make_async_copy(in_slice(0), in_v.at[b], gsem.at[b]).wait()

    def fire_out(it, b):
        pltpu.async_copy(out_v.at[b], out_hbm.at[pl.ds(it * BPD, BPD)],
                         osem.at[b])

    def wait_out(b):
        pltpu.make_async_copy(out_v.at[b], out_hbm.at[pl.ds(0, BPD)],
                              osem.at[b]).wait()

    def emit_block(b, a_count):
        iota = lax.iota(jnp.int32, 16)

        @plsc.parallel_loop(0, a_count, unroll=4)
        def row(a):
            for g in range(8):
                rows = iota + 16 * (g % 2)
                cols = jnp.full((16,), g // 2, jnp.int32) + a * 4
                vals = plsc.load_gather(in_v.at[b], [rows, cols])
                out_v[b, pl.ds(a * 128 + 16 * g, 16)] = vals

    def step(k, b):
        it = w + k * NW
        nxt = k + 1

        @pl.when(jnp.logical_or(nxt < FULL_K,
                                jnp.logical_and(nxt == FULL_K, w < 5)))
        def _():
            fire_in(w + nxt * NW, 1 - b)

        wait_in(b)

        @pl.when(k >= 2)
        def _():
            wait_out(b)

        emit_block(b, DIM)
        fire_out(it, b)

    fire_in(w, 0)

    def outer(j, carry):
        step(2 * j, 0)
        step(2 * j + 1, 1)
        return carry

    lax.fori_loop(0, FULL_K // 2, outer, 0)

    eb = FULL_K % 2
    e_it = w + FULL_K * NW

    @pl.when(w < 4)
    def _():
        wait_in(eb)
        wait_out(eb)
        emit_block(eb, DIM)
        fire_out(e_it, eb)
        wait_out(eb)

    @pl.when(w == 4)
    def _():
        wait_in(eb)
        wait_out(eb)
        emit_block(eb, 16)
        pltpu.async_copy(
            out_v.at[eb, pl.ds(0, BPD // 2)],
            out_hbm.at[pl.ds(e_it * BPD, BPD // 2)],
            osem.at[eb],
        )
        pltpu.make_async_copy(
            out_v.at[eb, pl.ds(0, BPD // 2)],
            out_hbm.at[pl.ds(0, BPD // 2)],
            osem.at[eb],
        ).wait()

    wait_out(1 - eb)

    @pl.when(w > 4)
    def _():
        wait_out(eb)


@functools.partial(
    pl.kernel,
    mesh=_mesh,
    out_type=jax.ShapeDtypeStruct((N, DIM), jnp.float32),
    scratch_types=[
        pltpu.VMEM((PER_W,), jnp.int32),
        pltpu.VMEM((NBUF, CHUNK, DIM), jnp.float32),
        pltpu.SemaphoreType.DMA((NBUF,)),
        pltpu.SemaphoreType.DMA((NBUF,)),
    ],
    compiler_params=pltpu.CompilerParams(use_tc_tiling_on_sc=False),
)
def _gather_kernel(idx_hbm, table_hbm, out_hbm, idx_v, rows_v, gsem, osem):
    base = _worker_id() * PER_W

    pltpu.sync_copy(idx_hbm.at[pl.ds(base, PER_W)], idx_v)

    def fire_gather(k, b):
        pltpu.async_copy(
            table_hbm.at[idx_v.at[pl.ds(k * CHUNK, CHUNK)]],
            rows_v.at[b],
            gsem.at[b],
        )

    for b in range(NBUF - 1):
        fire_gather(b, b)

    def step(i, b):
        pltpu.make_async_copy(table_hbm.at[idx_v.at[pl.ds(0, CHUNK)]],
                              rows_v.at[b], gsem.at[b]).wait()
        pltpu.async_copy(rows_v.at[b],
                         out_hbm.at[pl.ds(base + i * CHUNK, CHUNK)],
                         osem.at[b])
        nb = (b + NBUF - 1) % NBUF

        @pl.when(i + NBUF - 1 < NCHUNK)
        def _():
            @pl.when(i > 0)
            def _():
                pltpu.make_async_copy(
                    rows_v.at[nb],
                    out_hbm.at[pl.ds(base, CHUNK)],
                    osem.at[nb],
                ).wait()

            fire_gather(i + NBUF - 1, nb)

    def outer(j, carry):
        for b in range(NBUF):
            step(j * NBUF + b, b)
        return carry

    lax.fori_loop(0, NCHUNK // NBUF, outer, 0)

    for i in range(NCHUNK - NBUF, NCHUNK):
        b = i % NBUF
        pltpu.make_async_copy(
            rows_v.at[b],
            out_hbm.at[pl.ds(base, CHUNK)],
            osem.at[b],
        ).wait()


C_BLOCKS = BATCH // 1024
C_UNITS = HIST * C_BLOCKS
C_PER_W = C_UNITS // NW
C_IN = 1024 * DIM


@functools.partial(
    pl.kernel,
    mesh=_mesh,
    out_type=jax.ShapeDtypeStruct((HIST, DIM, BATCH), jnp.float32),
    scratch_types=[
        pltpu.VMEM((C_IN,), jnp.float32),
        pltpu.VMEM((C_IN,), jnp.float32),
        pltpu.VMEM((1, DIM, 1024), jnp.float32),
        pltpu.SemaphoreType.DMA((2,)),
        pltpu.SemaphoreType.DMA,
    ],
    compiler_params=pltpu.CompilerParams(needs_layout_passes=False),
)
def _final_kernel(flat_hbm, out_hbm, in_v0, in_v1, slab_v, gsem, osem):
    in_v = [in_v0, in_v1]
    w = _worker_id()
    u0 = w * C_PER_W
    iota32 = lax.iota(jnp.int32, 16) * DIM

    def unit_src(j):
        u = u0 + j
        h = u // C_BLOCKS
        b0 = (u % C_BLOCKS) * 1024
        return (h * BATCH + b0) * DIM

    def fire_in(j, b):
        pltpu.async_copy(flat_hbm.at[pl.ds(unit_src(j), C_IN)],
                         in_v[b], gsem.at[b])

    def wait_in(b):
        pltpu.make_async_copy(flat_hbm.at[pl.ds(0, C_IN)], in_v[b],
                              gsem.at[b]).wait()

    def out_piece(dt):
        return slab_v.at[pl.ds(0, 1), pl.ds(dt * 8, 8), pl.ds(0, 1024)]

    def wait_outs():
        for dt in range(4):
            pltpu.make_async_copy(
                out_piece(dt),
                out_hbm.at[pl.ds(0, 1), pl.ds(dt * 8, 8), pl.ds(0, 1024)],
                osem,
            ).wait()

    def transpose_unit(b):
        @plsc.parallel_loop(0, 1024 // 16, unroll=4)
        def col(g):
            base = g * (16 * DIM)
            for d in range(DIM):
                idx = iota32 + (base + d)
                vals = plsc.load_gather(in_v[b], [idx])
                slab_v[0, d, pl.ds(16 * g, 16)] = vals

    def step(j, b):
        @pl.when(j + 1 < C_PER_W)
        def _():
            fire_in(j + 1, 1 - b)

        wait_in(b)

        @pl.when(j > 0)
        def _():
            wait_outs()

        transpose_unit(b)
        u = u0 + j
        h = u // C_BLOCKS
        b0 = (u % C_BLOCKS) * 1024
        for dt in range(4):
            pltpu.async_copy(
                out_piece(dt),
                out_hbm.at[pl.ds(h, 1), pl.ds(dt * 8, 8), pl.ds(b0, 1024)],
                osem,
            )

    fire_in(0, 0)

    def outer(i, carry):
        step(2 * i, 0)
        step(2 * i + 1, 1)
        return carry

    lax.fori_loop(0, C_PER_W // 2, outer, 0)
    step(C_PER_W - 1, (C_PER_W - 1) % 2)
    wait_outs()


def kernel(x, table):
    idx = x.T.reshape(-1).astype(jnp.int32)
    table_rm = _transpose_kernel(table.T)
    rows = _gather_kernel(idx, table_rm.reshape(VOCAB, DIM))
    out3 = _final_kernel(rows.reshape(-1))
    return jnp.transpose(out3, (2, 0, 1))


# speedup vs baseline: 1.0597x; 1.0597x over previous
import functools

import jax
import jax.numpy as jnp
from jax import lax
from jax.experimental import pallas as pl
from jax.experimental.pallas import tpu as pltpu
from jax.experimental.pallas import tpu_sc as plsc

VOCAB = 1000000
DIM = 32
BATCH = 16384
HIST = 50
N = BATCH * HIST

NUM_CORES = 2
NUM_SUBCORES = 16
NW = NUM_CORES * NUM_SUBCORES

PER_W = N // NW
CHUNK = 800
NCHUNK = PER_W // CHUNK
NBUF = 4

NBLK = (VOCAB + 127) // 128
FULL_K = (NBLK - 1) // NW
BPD = 128 * DIM

_mesh = plsc.VectorSubcoreMesh(core_axis_name="c", subcore_axis_name="s")


def _worker_id():
    return lax.axis_index("s") * NUM_CORES + lax.axis_index("c")


@functools.partial(
    pl.kernel,
    mesh=_mesh,
    out_type=jax.ShapeDtypeStruct((VOCAB * DIM,), jnp.float32),
    scratch_types=[
        pltpu.VMEM((2, DIM, 128), jnp.float32),
        pltpu.VMEM((2, BPD), jnp.float32),
        pltpu.SemaphoreType.DMA((2,)),
        pltpu.SemaphoreType.DMA((2,)),
    ],
    compiler_params=pltpu.CompilerParams(needs_layout_passes=False),
)
def _transpose_kernel(tt_hbm, out_hbm, in_v, out_v, gsem, osem):
    w = _worker_id()

    def in_slice(it):
        return tt_hbm.at[pl.ds(0, DIM), pl.ds(it * 128, 128)]

    def fire_in(it, b):
        pltpu.async_copy(in_slice(it), in_v.at[b], gsem.at[b])

    def wait_in(b):
        pltpu.make_async_copy(in_slice(0), in_v.at[b], gsem.at[b]).wait()

    def fire_out(it, b):
        pltpu.async_copy(out_v.at[b], out_hbm.at[pl.ds(it * BPD, BPD)],
                         osem.at[b])

    def wait_out(b):
        pltpu.make_async_copy(out_v.at[b], out_hbm.at[pl.ds(0, BPD)],
                              osem.at[b]).wait()

    def emit_block(b, a_count):
        iota = lax.iota(jnp.int32, 16)

        @plsc.parallel_loop(0, a_count)
        def row(a):
            for g in range(8):
                rows = iota + 16 * (g % 2)
                cols = jnp.full((16,), g // 2, jnp.int32) + a * 4
                vals = plsc.load_gather(in_v.at[b], [rows, cols])
                out_v[b, pl.ds(a * 128 + 16 * g, 16)] = vals

    def step(k, b):
        it = w + k * NW
        nxt = k + 1

        @pl.when(jnp.logical_or(nxt < FULL_K,
                                jnp.logical_and(nxt == FULL_K, w < 5)))
        def _():
            fire_in(w + nxt * NW, 1 - b)

        wait_in(b)

        @pl.when(k >= 2)
        def _():
            wait_out(b)

        emit_block(b, DIM)
        fire_out(it, b)

    fire_in(w, 0)

    def outer(j, carry):
        step(2 * j, 0)
        step(2 * j + 1, 1)
        return carry

    lax.fori_loop(0, FULL_K // 2, outer, 0)

    eb = FULL_K % 2
    e_it = w + FULL_K * NW

    @pl.when(w < 4)
    def _():
        wait_in(eb)
        wait_out(eb)
        emit_block(eb, DIM)
        fire_out(e_it, eb)
        wait_out(eb)

    @pl.when(w == 4)
    def _():
        wait_in(eb)
        wait_out(eb)
        emit_block(eb, 16)
        pltpu.async_copy(
            out_v.at[eb, pl.ds(0, BPD // 2)],
            out_hbm.at[pl.ds(e_it * BPD, BPD // 2)],
            osem.at[eb],
        )
        pltpu.make_async_copy(
            out_v.at[eb, pl.ds(0, BPD // 2)],
            out_hbm.at[pl.ds(0, BPD // 2)],
            osem.at[eb],
        ).wait()

    wait_out(1 - eb)

    @pl.when(w > 4)
    def _():
        wait_out(eb)


@functools.partial(
    pl.kernel,
    mesh=_mesh,
    out_type=jax.ShapeDtypeStruct((N, DIM), jnp.float32),
    scratch_types=[
        pltpu.VMEM((PER_W,), jnp.int32),
        pltpu.VMEM((NBUF, CHUNK, DIM), jnp.float32),
        pltpu.SemaphoreType.DMA((NBUF,)),
        pltpu.SemaphoreType.DMA((NBUF,)),
    ],
    compiler_params=pltpu.CompilerParams(use_tc_tiling_on_sc=False),
)
def _gather_kernel(idx_hbm, table_hbm, out_hbm, idx_v, rows_v, gsem, osem):
    base = _worker_id() * PER_W

    pltpu.sync_copy(idx_hbm.at[pl.ds(base, PER_W)], idx_v)

    def fire_gather(k, b):
        pltpu.async_copy(
            table_hbm.at[idx_v.at[pl.ds(k * CHUNK, CHUNK)]],
            rows_v.at[b],
            gsem.at[b],
        )

    for b in range(NBUF - 1):
        fire_gather(b, b)

    def step(i, b):
        pltpu.make_async_copy(table_hbm.at[idx_v.at[pl.ds(0, CHUNK)]],
                              rows_v.at[b], gsem.at[b]).wait()
        pltpu.async_copy(rows_v.at[b],
                         out_hbm.at[pl.ds(base + i * CHUNK, CHUNK)],
                         osem.at[b])
        nb = (b + NBUF - 1) % NBUF

        @pl.when(i + NBUF - 1 < NCHUNK)
        def _():
            @pl.when(i > 0)
            def _():
                pltpu.make_async_copy(
                    rows_v.at[nb],
                    out_hbm.at[pl.ds(base, CHUNK)],
                    osem.at[nb],
                ).wait()

            fire_gather(i + NBUF - 1, nb)

    def outer(j, carry):
        for b in range(NBUF):
            step(j * NBUF + b, b)
        return carry

    lax.fori_loop(0, NCHUNK // NBUF, outer, 0)

    for i in range(NCHUNK - NBUF, NCHUNK):
        b = i % NBUF
        pltpu.make_async_copy(
            rows_v.at[b],
            out_hbm.at[pl.ds(base, CHUNK)],
            osem.at[b],
        ).wait()


C_BLOCKS = BATCH // 1024
C_UNITS = HIST * C_BLOCKS
C_PER_W = C_UNITS // NW
C_IN = 1024 * DIM


@functools.partial(
    pl.kernel,
    mesh=_mesh,
    out_type=jax.ShapeDtypeStruct((HIST, DIM, BATCH), jnp.float32),
    scratch_types=[
        pltpu.VMEM((C_IN,), jnp.float32),
        pltpu.VMEM((C_IN,), jnp.float32),
        pltpu.VMEM((1, DIM, 1024), jnp.float32),
        pltpu.SemaphoreType.DMA((2,)),
        pltpu.SemaphoreType.DMA,
    ],
    compiler_params=pltpu.CompilerParams(needs_layout_passes=False),
)
def _final_kernel(flat_hbm, out_hbm, in_v0, in_v1, slab_v, gsem, osem):
    in_v = [in_v0, in_v1]
    w = _worker_id()
    u0 = w * C_PER_W
    iota32 = lax.iota(jnp.int32, 16) * DIM

    def unit_src(j):
        u = u0 + j
        h = u // C_BLOCKS
        b0 = (u % C_BLOCKS) * 1024
        return (h * BATCH + b0) * DIM

    def fire_in(j, b):
        pltpu.async_copy(flat_hbm.at[pl.ds(unit_src(j), C_IN)],
                         in_v[b], gsem.at[b])

    def wait_in(b):
        pltpu.make_async_copy(flat_hbm.at[pl.ds(0, C_IN)], in_v[b],
                              gsem.at[b]).wait()

    def out_piece(dt):
        return slab_v.at[pl.ds(0, 1), pl.ds(dt * 8, 8), pl.ds(0, 1024)]

    def wait_outs():
        for dt in range(4):
            pltpu.make_async_copy(
                out_piece(dt),
                out_hbm.at[pl.ds(0, 1), pl.ds(dt * 8, 8), pl.ds(0, 1024)],
                osem,
            ).wait()

    def transpose_unit(b):
        @plsc.parallel_loop(0, 1024 // 16)
        def col(g):
            base = g * (16 * DIM)
            for d in range(DIM):
                idx = iota32 + (base + d)
                vals = plsc.load_gather(in_v[b], [idx])
                slab_v[0, d, pl.ds(16 * g, 16)] = vals

    def step(j, b):
        @pl.when(j + 1 < C_PER_W)
        def _():
            fire_in(j + 1, 1 - b)

        wait_in(b)

        @pl.when(j > 0)
        def _():
            wait_outs()

        transpose_unit(b)
        u = u0 + j
        h = u // C_BLOCKS
        b0 = (u % C_BLOCKS) * 1024
        for dt in range(4):
            pltpu.async_copy(
                out_piece(dt),
                out_hbm.at[pl.ds(h, 1), pl.ds(dt * 8, 8), pl.ds(b0, 1024)],
                osem,
            )

    fire_in(0, 0)

    def outer(i, carry):
        step(2 * i, 0)
        step(2 * i + 1, 1)
        return carry

    lax.fori_loop(0, C_PER_W // 2, outer, 0)
    step(C_PER_W - 1, (C_PER_W - 1) % 2)
    wait_outs()


def kernel(x, table):
    idx = x.T.reshape(-1).astype(jnp.int32)
    table_rm = _transpose_kernel(table.T)
    rows = _gather_kernel(idx, table_rm.reshape(VOCAB, DIM))
    out3 = _final_kernel(rows.reshape(-1))
    return jnp.transpose(out3, (2, 0, 1))


# speedup vs baseline: 1.0609x; 1.0012x over previous
import functools

import jax
import jax.numpy as jnp
from jax import lax
from jax.experimental import pallas as pl
from jax.experimental.pallas import tpu as pltpu
from jax.experimental.pallas import tpu_sc as plsc

VOCAB = 1000000
DIM = 32
BATCH = 16384
HIST = 50
N = BATCH * HIST

NUM_CORES = 2
NUM_SUBCORES = 16
NW = NUM_CORES * NUM_SUBCORES

PER_W = N // NW
CHUNK = 800
NCHUNK = PER_W // CHUNK
NBUF = 4

ABW = 256
NFULL = VOCAB // ABW
FULL_K = NFULL // NW
NEXTRA = NFULL - FULL_K * NW
BPD = ABW * DIM
TAILB = 2048

_mesh = plsc.VectorSubcoreMesh(core_axis_name="c", subcore_axis_name="s")


def _worker_id():
    return lax.axis_index("s") * NUM_CORES + lax.axis_index("c")


@functools.partial(
    pl.kernel,
    mesh=_mesh,
    out_type=jax.ShapeDtypeStruct((VOCAB * DIM,), jnp.float32),
    scratch_types=[
        pltpu.VMEM((2, DIM, ABW), jnp.float32),
        pltpu.VMEM((2, BPD), jnp.float32),
        pltpu.SemaphoreType.DMA((2,)),
        pltpu.SemaphoreType.DMA((2,)),
    ],
    compiler_params=pltpu.CompilerParams(needs_layout_passes=False),
)
def _transpose_kernel(tt_hbm, out_hbm, in_v, out_v, gsem, osem):
    w = _worker_id()

    def in_slice(it):
        return tt_hbm.at[pl.ds(0, DIM), pl.ds(it * ABW, ABW)]

    def fire_in(it, b):
        pltpu.async_copy(in_slice(it), in_v.at[b], gsem.at[b])

    def wait_in(b):
        pltpu.make_async_copy(in_slice(0), in_v.at[b], gsem.at[b]).wait()

    def fire_out(it, b):
        pltpu.async_copy(out_v.at[b], out_hbm.at[pl.ds(it * BPD, BPD)],
                         osem.at[b])

    def wait_out(b):
        pltpu.make_async_copy(out_v.at[b], out_hbm.at[pl.ds(0, BPD)],
                              osem.at[b]).wait()

    def emit_block(b, a_count):
        iota = lax.iota(jnp.int32, 16)

        @plsc.parallel_loop(0, a_count)
        def row(a):
            for g in range(8):
                rows = iota + 16 * (g % 2)
                cols = jnp.full((16,), g // 2, jnp.int32) + a * 4
                vals = plsc.load_gather(in_v.at[b], [rows, cols])
                out_v[b, pl.ds(a * 128 + 16 * g, 16)] = vals

    def step(k, b):
        it = w + k * NW
        nxt = k + 1

        @pl.when(jnp.logical_or(nxt < FULL_K,
                                jnp.logical_and(nxt == FULL_K, w < NEXTRA)))
        def _():
            fire_in(w + nxt * NW, 1 - b)

        @pl.when(jnp.logical_and(nxt == FULL_K, w == NEXTRA))
        def _():
            pltpu.async_copy(
                tt_hbm.at[pl.ds(0, DIM),
                          pl.ds(NFULL * ABW + (w - NEXTRA) * 128, 128)],
                in_v.at[1 - b, pl.ds(0, DIM), pl.ds(0, 128)],
                gsem.at[1 - b],
            )

        wait_in(b)

        @pl.when(k >= 2)
        def _():
            wait_out(b)

        emit_block(b, BPD // 128)
        fire_out(it, b)

    fire_in(w, 0)

    def outer(j, carry):
        step(2 * j, 0)
        step(2 * j + 1, 1)
        return carry

    lax.fori_loop(0, FULL_K // 2, outer, 0)

    eb = FULL_K % 2
    e_it = w + FULL_K * NW

    @pl.when(w < NEXTRA)
    def _():
        wait_in(eb)
        wait_out(eb)
        emit_block(eb, BPD // 128)
        fire_out(e_it, eb)
        wait_out(eb)

    @pl.when(w == NEXTRA)
    def _():
        pltpu.make_async_copy(
            tt_hbm.at[pl.ds(0, DIM), pl.ds(0, 128)],
            in_v.at[eb, pl.ds(0, DIM), pl.ds(0, 128)],
            gsem.at[eb],
        ).wait()
        wait_out(eb)
        emit_block(eb, 16)
        pltpu.async_copy(
            out_v.at[eb, pl.ds(0, TAILB)],
            out_hbm.at[pl.ds(NFULL * BPD, TAILB)],
            osem.at[eb],
        )
        pltpu.make_async_copy(
            out_v.at[eb, pl.ds(0, TAILB)],
            out_hbm.at[pl.ds(0, TAILB)],
            osem.at[eb],
        ).wait()

    wait_out(1 - eb)

    @pl.when(w > NEXTRA)
    def _():
        wait_out(eb)


@functools.partial(
    pl.kernel,
    mesh=_mesh,
    out_type=jax.ShapeDtypeStruct((N, DIM), jnp.float32),
    scratch_types=[
        pltpu.VMEM((PER_W,), jnp.int32),
        pltpu.VMEM((NBUF, CHUNK, DIM), jnp.float32),
        pltpu.SemaphoreType.DMA((NBUF,)),
        pltpu.SemaphoreType.DMA((NBUF,)),
    ],
    compiler_params=pltpu.CompilerParams(use_tc_tiling_on_sc=False),
)
def _gather_kernel(idx_hbm, table_hbm, out_hbm, idx_v, rows_v, gsem, osem):
    base = _worker_id() * PER_W

    pltpu.sync_copy(idx_hbm.at[pl.ds(base, PER_W)], idx_v)

    def fire_gather(k, b):
        pltpu.async_copy(
            table_hbm.at[idx_v.at[pl.ds(k * CHUNK, CHUNK)]],
            rows_v.at[b],
            gsem.at[b],
        )

    for b in range(NBUF - 1):
        fire_gather(b, b)

    def step(i, b):
        pltpu.make_async_copy(table_hbm.at[idx_v.at[pl.ds(0, CHUNK)]],
                              rows_v.at[b], gsem.at[b]).wait()
        pltpu.async_copy(rows_v.at[b],
                         out_hbm.at[pl.ds(base + i * CHUNK, CHUNK)],
                         osem.at[b])
        nb = (b + NBUF - 1) % NBUF

        @pl.when(i + NBUF - 1 < NCHUNK)
        def _():
            @pl.when(i > 0)
            def _():
                pltpu.make_async_copy(
                    rows_v.at[nb],
                    out_hbm.at[pl.ds(base, CHUNK)],
                    osem.at[nb],
                ).wait()

            fire_gather(i + NBUF - 1, nb)

    def outer(j, carry):
        for b in range(NBUF):
            step(j * NBUF + b, b)
        return carry

    lax.fori_loop(0, NCHUNK // NBUF, outer, 0)

    for i in range(NCHUNK - NBUF, NCHUNK):
        b = i % NBUF
        pltpu.make_async_copy(
            rows_v.at[b],
            out_hbm.at[pl.ds(base, CHUNK)],
            osem.at[b],
        ).wait()


C_BLOCKS = BATCH // 1024
C_UNITS = HIST * C_BLOCKS
C_PER_W = C_UNITS // NW
C_IN = 1024 * DIM


@functools.partial(
    pl.kernel,
    mesh=_mesh,
    out_type=jax.ShapeDtypeStruct((HIST, DIM, BATCH), jnp.float32),
    scratch_types=[
        pltpu.VMEM((C_IN,), jnp.float32),
        pltpu.VMEM((C_IN,), jnp.float32),
        pltpu.VMEM((1, DIM, 1024), jnp.float32),
        pltpu.SemaphoreType.DMA((2,)),
        pltpu.SemaphoreType.DMA,
    ],
    compiler_params=pltpu.CompilerParams(needs_layout_passes=False),
)
def _final_kernel(flat_hbm, out_hbm, in_v0, in_v1, slab_v, gsem, osem):
    in_v = [in_v0, in_v1]
    w = _worker_id()
    u0 = w * C_PER_W
    iota32 = lax.iota(jnp.int32, 16) * DIM

    def unit_src(j):
        u = u0 + j
        h = u // C_BLOCKS
        b0 = (u % C_BLOCKS) * 1024
        return (h * BATCH + b0) * DIM

    def fire_in(j, b):
        pltpu.async_copy(flat_hbm.at[pl.ds(unit_src(j), C_IN)],
                         in_v[b], gsem.at[b])

    def wait_in(b):
        pltpu.make_async_copy(flat_hbm.at[pl.ds(0, C_IN)], in_v[b],
                              gsem.at[b]).wait()

    def out_piece(dt):
        return slab_v.at[pl.ds(0, 1), pl.ds(dt * 8, 8), pl.ds(0, 1024)]

    def wait_outs():
        for dt in range(4):
            pltpu.make_async_copy(
                out_piece(dt),
                out_hbm.at[pl.ds(0, 1), pl.ds(dt * 8, 8), pl.ds(0, 1024)],
                osem,
            ).wait()

    def transpose_unit(b):
        @plsc.parallel_loop(0, 1024 // 16)
        def col(g):
            base = g * (16 * DIM)
            for d in range(DIM):
                idx = iota32 + (base + d)
                vals = plsc.load_gather(in_v[b], [idx])
                slab_v[0, d, pl.ds(16 * g, 16)] = vals

    def step(j, b):
        @pl.when(j + 1 < C_PER_W)
        def _():
            fire_in(j + 1, 1 - b)

        wait_in(b)

        @pl.when(j > 0)
        def _():
            wait_outs()

        transpose_unit(b)
        u = u0 + j
        h = u // C_BLOCKS
        b0 = (u % C_BLOCKS) * 1024
        for dt in range(4):
            pltpu.async_copy(
                out_piece(dt),
                out_hbm.at[pl.ds(h, 1), pl.ds(dt * 8, 8), pl.ds(b0, 1024)],
                osem,
            )

    fire_in(0, 0)

    def outer(i, carry):
        step(2 * i, 0)
        step(2 * i + 1, 1)
        return carry

    lax.fori_loop(0, C_PER_W // 2, outer, 0)
    step(C_PER_W - 1, (C_PER_W - 1) % 2)
    wait_outs()


def kernel(x, table):
    idx = x.T.reshape(-1).astype(jnp.int32)
    table_rm = _transpose_kernel(table.T)
    rows = _gather_kernel(idx, table_rm.reshape(VOCAB, DIM))
    out3 = _final_kernel(rows.reshape(-1))
    return jnp.transpose(out3, (2, 0, 1))
